# baseline (speedup 1.0000x reference)
import jax
import jax.numpy as jnp
import numpy as np
from jax import lax
from jax.experimental import pallas as pl
from jax.experimental.pallas import tpu as pltpu

B, T, E = 1, 2048, 768
H, D = 12, 64
NE, ES, A = 8, 256, 2

BT = 256
NT = T // BT


def _rope_apply(y, cos, ssin):
    d = lax.broadcasted_iota(jnp.int32, y.shape, 1) % D
    first = d < (D // 2)
    left = jnp.concatenate([y[:, D // 2:], y[:, : D // 2]], axis=1)
    right = jnp.concatenate([y[:, -(D // 2):], y[:, : -(D // 2)]], axis=1)
    partner = jnp.where(first, left, right)
    return y * cos + partner * ssin


def _heads(y):
    return jnp.stack([y[:, h * D:(h + 1) * D] for h in range(H)], axis=0)


def _proj_kernel(x_ref, w_ref, cos_ref, ssin_ref,
                 qf_ref, qa_ref, ka_ref, va_ref, gates_ref):
    x = x_ref[...]
    h = lax.dot_general(x, w_ref[...], (((1,), (1,)), ((), ())),
                        preferred_element_type=jnp.float32)
    qf_ref[...] = h[:, :E]
    cos = cos_ref[...]
    ssin = ssin_ref[...]
    qa_ref[...] = _heads(_rope_apply(h[:, E:2 * E], cos, ssin))
    ka_ref[...] = _heads(_rope_apply(h[:, 2 * E:3 * E], cos, ssin))
    va_ref[...] = _heads(h[:, 3 * E:4 * E])
    logits = h[:, 4 * E:4 * E + NE]
    col = lax.broadcasted_iota(jnp.int32, (BT, NE), 1)
    cols = []
    for n in range(NE):
        ln = logits[:, n:n + 1]
        greater = jnp.sum((logits > ln).astype(jnp.float32), axis=1,
                          keepdims=True)
        eq_before = jnp.sum(((logits == ln) & (col < n)).astype(jnp.float32),
                            axis=1, keepdims=True)
        rank = greater + eq_before
        cols.append(jnp.where(rank < A, jax.nn.sigmoid(ln), 0.0))
    gates_ref[...] = jnp.concatenate(cols, axis=1)


def _attn_kernel(q_ref, k_ref, v_ref, o_ref, m_ref, l_ref, acc_ref):
    qi = pl.program_id(1)
    j = pl.program_id(2)

    @pl.when(j == 0)
    def _init():
        m_ref[...] = jnp.full_like(m_ref[...], -1e30)
        l_ref[...] = jnp.zeros_like(l_ref[...])
        acc_ref[...] = jnp.zeros_like(acc_ref[...])

    @pl.when(j <= qi)
    def _step():
        q = q_ref[0]
        k = k_ref[0]
        s = lax.dot_general(q, k, (((1,), (1,)), ((), ())),
                            preferred_element_type=jnp.float32)
        s = s * (1.0 / np.sqrt(D))
        row = qi * BT + lax.broadcasted_iota(jnp.int32, s.shape, 0)
        ccol = j * BT + lax.broadcasted_iota(jnp.int32, s.shape, 1)
        s = jnp.where(ccol <= row, s, -1e30)
        m_prev = m_ref[...]
        m_new = jnp.maximum(m_prev, jnp.max(s, axis=1, keepdims=True))
        corr = jnp.exp(m_prev - m_new)
        p = jnp.exp(s - m_new)
        l_ref[...] = l_ref[...] * corr + jnp.sum(p, axis=1, keepdims=True)
        acc_ref[...] = acc_ref[...] * corr + jnp.dot(
            p, v_ref[0], preferred_element_type=jnp.float32)
        m_ref[...] = m_new

    @pl.when(j == qi)
    def _finalize():
        o_ref[0] = acc_ref[...] / l_ref[...]


def _moe_out_kernel(qf_ref, gates_ref, attn_ref, kf_ref, vf_ref, w_ref,
                    o_ref):
    gates = gates_ref[...]
    expand = (lax.broadcasted_iota(jnp.int32, (NE, NE * ES), 0) ==
              lax.broadcasted_iota(jnp.int32, (NE, NE * ES), 1) // ES)
    ge = jnp.dot(gates, expand.astype(jnp.float32),
                 preferred_element_type=jnp.float32)
    ffwd_cols = []
    for h in range(H):
        qh = qf_ref[:, h * D:(h + 1) * D]
        s = lax.dot_general(qh, kf_ref[h], (((1,), (1,)), ((), ())),
                            preferred_element_type=jnp.float32)
        a = 0.5 * s * (1.0 + lax.erf(s * np.float32(1.0 / np.sqrt(2.0))))
        ffwd_cols.append(jnp.dot(a * ge, vf_ref[h],
                                 preferred_element_type=jnp.float32))
    ffwd = jnp.concatenate(ffwd_cols, axis=1)
    attn = jnp.concatenate([attn_ref[h] for h in range(H)], axis=1)
    w = w_ref[...]
    out = lax.dot_general(attn, w[:, :E], (((1,), (1,)), ((), ())),
                          preferred_element_type=jnp.float32)
    out += lax.dot_general(ffwd, w[:, E:], (((1,), (1,)), ((), ())),
                           preferred_element_type=jnp.float32)
    o_ref[...] = out


@jax.jit
def kernel(x, W_in, W_out, k_ffwd, v_ffwd):
    x2 = x.reshape(T, E)
    pos = np.arange(T, dtype=np.float32)
    dh = np.arange(E) % D
    inv_freq = (1.0 / (10000.0 ** (np.arange(0, D, 2, dtype=np.float32) / D)))
    ang = pos[:, None] * inv_freq[dh % (D // 2)][None, :]
    cos_t = jnp.asarray(np.cos(ang), dtype=jnp.float32)
    ssin_t = jnp.asarray(np.sin(ang) * np.where(dh < D // 2, -1.0, 1.0),
                         dtype=jnp.float32)

    qf, qa3, ka3, va3, gates = pl.pallas_call(
        _proj_kernel,
        grid=(NT,),
        in_specs=[
            pl.BlockSpec((BT, E), lambda i: (i, 0)),
            pl.BlockSpec((4 * E + NE, E), lambda i: (0, 0)),
            pl.BlockSpec((BT, E), lambda i: (i, 0)),
            pl.BlockSpec((BT, E), lambda i: (i, 0)),
        ],
        out_specs=[
            pl.BlockSpec((BT, E), lambda i: (i, 0)),
            pl.BlockSpec((H, BT, D), lambda i: (0, i, 0)),
            pl.BlockSpec((H, BT, D), lambda i: (0, i, 0)),
            pl.BlockSpec((H, BT, D), lambda i: (0, i, 0)),
            pl.BlockSpec((BT, NE), lambda i: (i, 0)),
        ],
        out_shape=[
            jax.ShapeDtypeStruct((T, E), jnp.float32),
            jax.ShapeDtypeStruct((H, T, D), jnp.float32),
            jax.ShapeDtypeStruct((H, T, D), jnp.float32),
            jax.ShapeDtypeStruct((H, T, D), jnp.float32),
            jax.ShapeDtypeStruct((T, NE), jnp.float32),
        ],
    )(x2, W_in, cos_t, ssin_t)

    attn3 = pl.pallas_call(
        _attn_kernel,
        grid=(H, NT, NT),
        in_specs=[
            pl.BlockSpec((1, BT, D), lambda h, qi, j: (h, qi, 0)),
            pl.BlockSpec((1, BT, D), lambda h, qi, j: (h, j, 0)),
            pl.BlockSpec((1, BT, D), lambda h, qi, j: (h, j, 0)),
        ],
        out_specs=pl.BlockSpec((1, BT, D), lambda h, qi, j: (h, qi, 0)),
        out_shape=jax.ShapeDtypeStruct((H, T, D), jnp.float32),
        scratch_shapes=[
            pltpu.VMEM((BT, 1), jnp.float32),
            pltpu.VMEM((BT, 1), jnp.float32),
            pltpu.VMEM((BT, D), jnp.float32),
        ],
    )(qa3, ka3, va3)

    kf2 = k_ffwd.reshape(H, NE * ES, D)
    vf2 = v_ffwd.reshape(H, NE * ES, D)
    out = pl.pallas_call(
        _moe_out_kernel,
        grid=(NT,),
        in_specs=[
            pl.BlockSpec((BT, E), lambda i: (i, 0)),
            pl.BlockSpec((BT, NE), lambda i: (i, 0)),
            pl.BlockSpec((H, BT, D), lambda i: (0, i, 0)),
            pl.BlockSpec((H, NE * ES, D), lambda i: (0, 0, 0)),
            pl.BlockSpec((H, NE * ES, D), lambda i: (0, 0, 0)),
            pl.BlockSpec((E, 2 * E), lambda i: (0, 0)),
        ],
        out_specs=pl.BlockSpec((BT, E), lambda i: (i, 0)),
        out_shape=jax.ShapeDtypeStruct((T, E), jnp.float32),
    )(qf, gates, attn3, kf2, vf2, W_out)

    return out.reshape(B, T, E)


# speedup vs baseline: 2.0976x; 2.0976x over previous
import jax
import jax.numpy as jnp
import numpy as np
from jax import lax
from jax.experimental import pallas as pl
from jax.experimental.pallas import tpu as pltpu

B, T, E = 1, 2048, 768
H, D = 12, 64
NE, ES, A = 8, 256, 2

BT = 256
NT = T // BT


def _rope_apply(y, cos, ssin):
    d = lax.broadcasted_iota(jnp.int32, y.shape, 1) % D
    first = d < (D // 2)
    left = jnp.concatenate([y[:, D // 2:], y[:, : D // 2]], axis=1)
    right = jnp.concatenate([y[:, -(D // 2):], y[:, : -(D // 2)]], axis=1)
    partner = jnp.where(first, left, right)
    return y * cos + partner * ssin


def _heads(y):
    return jnp.stack([y[:, h * D:(h + 1) * D] for h in range(H)], axis=0)


def _proj_kernel(x_ref, w_ref, cos_ref, ssin_ref,
                 qf_ref, qa_ref, ka_ref, va_ref, gates_ref):
    x = x_ref[...]
    h = lax.dot_general(x, w_ref[...], (((1,), (1,)), ((), ())),
                        preferred_element_type=jnp.float32)
    qf_ref[...] = h[:, :E]
    cos = cos_ref[...]
    ssin = ssin_ref[...]
    qa_ref[...] = _heads(_rope_apply(h[:, E:2 * E], cos, ssin))
    ka_ref[...] = _heads(_rope_apply(h[:, 2 * E:3 * E], cos, ssin))
    va_ref[...] = _heads(h[:, 3 * E:4 * E])
    logits = h[:, 4 * E:4 * E + NE]
    col = lax.broadcasted_iota(jnp.int32, (BT, NE), 1)
    cols = []
    for n in range(NE):
        ln = logits[:, n:n + 1]
        greater = jnp.sum((logits > ln).astype(jnp.float32), axis=1,
                          keepdims=True)
        eq_before = jnp.sum(((logits == ln) & (col < n)).astype(jnp.float32),
                            axis=1, keepdims=True)
        rank = greater + eq_before
        cols.append(jnp.where(rank < A, jax.nn.sigmoid(ln), 0.0))
    gates_ref[...] = jnp.concatenate(cols, axis=1)


def _attn_kernel(q_ref, k_ref, v_ref, o_ref):
    qi = pl.program_id(1)
    q = q_ref[0]
    k = k_ref[0]
    s = lax.dot_general(q, k, (((1,), (1,)), ((), ())),
                        preferred_element_type=jnp.float32)
    s = s * (1.0 / np.sqrt(D))
    row = qi * BT + lax.broadcasted_iota(jnp.int32, s.shape, 0)
    ccol = lax.broadcasted_iota(jnp.int32, s.shape, 1)
    s = jnp.where(ccol <= row, s, -1e30)
    m = jnp.max(s, axis=1, keepdims=True)
    p = jnp.exp(s - m)
    p = p / jnp.sum(p, axis=1, keepdims=True)
    o_ref[0] = jnp.dot(p, v_ref[0], preferred_element_type=jnp.float32)


def _moe_out_kernel(qf_ref, gates_ref, attn_ref, kf_ref, vf_ref, w_ref,
                    o_ref):
    gates = gates_ref[...]
    expand = (lax.broadcasted_iota(jnp.int32, (NE, NE * ES), 0) ==
              lax.broadcasted_iota(jnp.int32, (NE, NE * ES), 1) // ES)
    ge = jnp.dot(gates, expand.astype(jnp.float32),
                 preferred_element_type=jnp.float32)
    ffwd_cols = []
    for h in range(H):
        qh = qf_ref[:, h * D:(h + 1) * D]
        s = lax.dot_general(qh, kf_ref[h], (((1,), (1,)), ((), ())),
                            preferred_element_type=jnp.float32)
        a = 0.5 * s * (1.0 + lax.erf(s * np.float32(1.0 / np.sqrt(2.0))))
        ffwd_cols.append(jnp.dot(a * ge, vf_ref[h],
                                 preferred_element_type=jnp.float32))
    ffwd = jnp.concatenate(ffwd_cols, axis=1)
    attn = jnp.concatenate([attn_ref[h] for h in range(H)], axis=1)
    w = w_ref[...]
    out = lax.dot_general(attn, w[:, :E], (((1,), (1,)), ((), ())),
                          preferred_element_type=jnp.float32)
    out += lax.dot_general(ffwd, w[:, E:], (((1,), (1,)), ((), ())),
                           preferred_element_type=jnp.float32)
    o_ref[...] = out


@jax.jit
def kernel(x, W_in, W_out, k_ffwd, v_ffwd):
    x2 = x.reshape(T, E)
    pos = np.arange(T, dtype=np.float32)
    dh = np.arange(E) % D
    inv_freq = (1.0 / (10000.0 ** (np.arange(0, D, 2, dtype=np.float32) / D)))
    ang = pos[:, None] * inv_freq[dh % (D // 2)][None, :]
    cos_t = jnp.asarray(np.cos(ang), dtype=jnp.float32)
    ssin_t = jnp.asarray(np.sin(ang) * np.where(dh < D // 2, -1.0, 1.0),
                         dtype=jnp.float32)

    qf, qa3, ka3, va3, gates = pl.pallas_call(
        _proj_kernel,
        grid=(NT,),
        in_specs=[
            pl.BlockSpec((BT, E), lambda i: (i, 0)),
            pl.BlockSpec((4 * E + NE, E), lambda i: (0, 0)),
            pl.BlockSpec((BT, E), lambda i: (i, 0)),
            pl.BlockSpec((BT, E), lambda i: (i, 0)),
        ],
        out_specs=[
            pl.BlockSpec((BT, E), lambda i: (i, 0)),
            pl.BlockSpec((H, BT, D), lambda i: (0, i, 0)),
            pl.BlockSpec((H, BT, D), lambda i: (0, i, 0)),
            pl.BlockSpec((H, BT, D), lambda i: (0, i, 0)),
            pl.BlockSpec((BT, NE), lambda i: (i, 0)),
        ],
        out_shape=[
            jax.ShapeDtypeStruct((T, E), jnp.float32),
            jax.ShapeDtypeStruct((H, T, D), jnp.float32),
            jax.ShapeDtypeStruct((H, T, D), jnp.float32),
            jax.ShapeDtypeStruct((H, T, D), jnp.float32),
            jax.ShapeDtypeStruct((T, NE), jnp.float32),
        ],
    )(x2, W_in, cos_t, ssin_t)

    attn3 = pl.pallas_call(
        _attn_kernel,
        grid=(H, NT),
        in_specs=[
            pl.BlockSpec((1, BT, D), lambda h, qi: (h, qi, 0)),
            pl.BlockSpec((1, T, D), lambda h, qi: (h, 0, 0)),
            pl.BlockSpec((1, T, D), lambda h, qi: (h, 0, 0)),
        ],
        out_specs=pl.BlockSpec((1, BT, D), lambda h, qi: (h, qi, 0)),
        out_shape=jax.ShapeDtypeStruct((H, T, D), jnp.float32),
    )(qa3, ka3, va3)

    kf2 = k_ffwd.reshape(H, NE * ES, D)
    vf2 = v_ffwd.reshape(H, NE * ES, D)
    out = pl.pallas_call(
        _moe_out_kernel,
        grid=(NT,),
        in_specs=[
            pl.BlockSpec((BT, E), lambda i: (i, 0)),
            pl.BlockSpec((BT, NE), lambda i: (i, 0)),
            pl.BlockSpec((H, BT, D), lambda i: (0, i, 0)),
            pl.BlockSpec((H, NE * ES, D), lambda i: (0, 0, 0)),
            pl.BlockSpec((H, NE * ES, D), lambda i: (0, 0, 0)),
            pl.BlockSpec((E, 2 * E), lambda i: (0, 0)),
        ],
        out_specs=pl.BlockSpec((BT, E), lambda i: (i, 0)),
        out_shape=jax.ShapeDtypeStruct((T, E), jnp.float32),
    )(qf, gates, attn3, kf2, vf2, W_out)

    return out.reshape(B, T, E)


# speedup vs baseline: 2.4965x; 1.1901x over previous
import jax
import jax.numpy as jnp
import numpy as np
from jax import lax
from jax.experimental import pallas as pl
from jax.experimental.pallas import tpu as pltpu

B, T, E = 1, 2048, 768
H, D = 12, 64
NE, ES, A = 8, 256, 2

BT = 256
NT = T // BT
BF = jnp.bfloat16
F32 = jnp.float32


def _rope_apply(y, cos, ssin):
    d = lax.broadcasted_iota(jnp.int32, y.shape, 1) % D
    first = d < (D // 2)
    left = jnp.concatenate([y[:, D // 2:], y[:, : D // 2]], axis=1)
    right = jnp.concatenate([y[:, -(D // 2):], y[:, : -(D // 2)]], axis=1)
    partner = jnp.where(first, left, right)
    return y * cos + partner * ssin


def _heads(y):
    return jnp.stack([y[:, h * D:(h + 1) * D] for h in range(H)], axis=0)


def _proj_kernel(x_ref, w_ref, wr_ref, cos_ref, ssin_ref,
                 qf_ref, qa_ref, ka_ref, va_ref, gates_ref):
    x = x_ref[...]
    xb = x.astype(BF)
    h = lax.dot_general(xb, w_ref[...], (((1,), (1,)), ((), ())),
                        preferred_element_type=F32)
    qf_ref[...] = h[:, :E].astype(BF)
    cos = cos_ref[...]
    ssin = ssin_ref[...]
    qa_ref[...] = _heads(_rope_apply(h[:, E:2 * E], cos, ssin).astype(BF))
    ka_ref[...] = _heads(_rope_apply(h[:, 2 * E:3 * E], cos, ssin).astype(BF))
    va_ref[...] = _heads(h[:, 3 * E:4 * E].astype(BF))
    logits = lax.dot_general(x, wr_ref[...], (((1,), (1,)), ((), ())),
                             preferred_element_type=F32)
    col = lax.broadcasted_iota(jnp.int32, (BT, NE), 1)
    rank = jnp.zeros((BT, NE), dtype=F32)
    for j in range(NE):
        lj = logits[:, j:j + 1]
        rank += (lj > logits).astype(F32)
        rank += ((lj == logits) & (col > j)).astype(F32)
    gates_ref[...] = jnp.where(rank < A, jax.nn.sigmoid(logits), 0.0)


def _attn_kernel(q_ref, k_ref, v_ref, o_ref):
    qi = pl.program_id(1)
    q = q_ref[0]
    k = k_ref[0]
    s = lax.dot_general(q, k, (((1,), (1,)), ((), ())),
                        preferred_element_type=F32)
    s = s * (1.0 / np.sqrt(D))
    row = qi * BT + lax.broadcasted_iota(jnp.int32, s.shape, 0)
    ccol = lax.broadcasted_iota(jnp.int32, s.shape, 1)
    s = jnp.where(ccol <= row, s, -1e30)
    m = jnp.max(s, axis=1, keepdims=True)
    p = jnp.exp(s - m)
    p = p / jnp.sum(p, axis=1, keepdims=True)
    o_ref[0] = jnp.dot(p.astype(BF), v_ref[0],
                       preferred_element_type=F32).astype(BF)


def _moe_out_kernel(qf_ref, gates_ref, attn_ref, kf_ref, vf_ref, w_ref,
                    o_ref):
    gates = gates_ref[...]
    expand = (lax.broadcasted_iota(jnp.int32, (NE, NE * ES), 0) ==
              lax.broadcasted_iota(jnp.int32, (NE, NE * ES), 1) // ES)
    ge = jnp.dot(gates, expand.astype(F32), preferred_element_type=F32)
    ffwd_cols = []
    for h in range(H):
        qh = qf_ref[:, h * D:(h + 1) * D]
        s = lax.dot_general(qh, kf_ref[h], (((1,), (1,)), ((), ())),
                            preferred_element_type=F32)
        a = 0.5 * s * (1.0 + lax.erf(s * np.float32(1.0 / np.sqrt(2.0))))
        ffwd_cols.append(jnp.dot((a * ge).astype(BF), vf_ref[h],
                                 preferred_element_type=F32))
    ffwd = jnp.concatenate(ffwd_cols, axis=1).astype(BF)
    attn = jnp.concatenate([attn_ref[h] for h in range(H)], axis=1)
    w = w_ref[...]
    out = lax.dot_general(attn, w[:, :E], (((1,), (1,)), ((), ())),
                          preferred_element_type=F32)
    out += lax.dot_general(ffwd, w[:, E:], (((1,), (1,)), ((), ())),
                           preferred_element_type=F32)
    o_ref[...] = out


@jax.jit
def kernel(x, W_in, W_out, k_ffwd, v_ffwd):
    x2 = x.reshape(T, E)
    pos = np.arange(T, dtype=np.float32)
    dh = np.arange(E) % D
    inv_freq = (1.0 / (10000.0 ** (np.arange(0, D, 2, dtype=np.float32) / D)))
    ang = pos[:, None] * inv_freq[dh % (D // 2)][None, :]
    cos_t = jnp.asarray(np.cos(ang), dtype=F32)
    ssin_t = jnp.asarray(np.sin(ang) * np.where(dh < D // 2, -1.0, 1.0),
                         dtype=F32)

    w_main = W_in[:4 * E].astype(BF)
    w_r = W_in[4 * E:]

    qf, qa3, ka3, va3, gates = pl.pallas_call(
        _proj_kernel,
        grid=(NT,),
        in_specs=[
            pl.BlockSpec((BT, E), lambda i: (i, 0)),
            pl.BlockSpec((4 * E, E), lambda i: (0, 0)),
            pl.BlockSpec((NE, E), lambda i: (0, 0)),
            pl.BlockSpec((BT, E), lambda i: (i, 0)),
            pl.BlockSpec((BT, E), lambda i: (i, 0)),
        ],
        out_specs=[
            pl.BlockSpec((BT, E), lambda i: (i, 0)),
            pl.BlockSpec((H, BT, D), lambda i: (0, i, 0)),
            pl.BlockSpec((H, BT, D), lambda i: (0, i, 0)),
            pl.BlockSpec((H, BT, D), lambda i: (0, i, 0)),
            pl.BlockSpec((BT, NE), lambda i: (i, 0)),
        ],
        out_shape=[
            jax.ShapeDtypeStruct((T, E), BF),
            jax.ShapeDtypeStruct((H, T, D), BF),
            jax.ShapeDtypeStruct((H, T, D), BF),
            jax.ShapeDtypeStruct((H, T, D), BF),
            jax.ShapeDtypeStruct((T, NE), F32),
        ],
    )(x2, w_main, w_r, cos_t, ssin_t)

    attn3 = pl.pallas_call(
        _attn_kernel,
        grid=(H, NT),
        in_specs=[
            pl.BlockSpec((1, BT, D), lambda h, qi: (h, qi, 0)),
            pl.BlockSpec((1, T, D), lambda h, qi: (h, 0, 0)),
            pl.BlockSpec((1, T, D), lambda h, qi: (h, 0, 0)),
        ],
        out_specs=pl.BlockSpec((1, BT, D), lambda h, qi: (h, qi, 0)),
        out_shape=jax.ShapeDtypeStruct((H, T, D), BF),
    )(qa3, ka3, va3)

    kf2 = k_ffwd.reshape(H, NE * ES, D).astype(BF)
    vf2 = v_ffwd.reshape(H, NE * ES, D).astype(BF)
    wout_bf = W_out.astype(BF)
    out = pl.pallas_call(
        _moe_out_kernel,
        grid=(NT,),
        in_specs=[
            pl.BlockSpec((BT, E), lambda i: (i, 0)),
            pl.BlockSpec((BT, NE), lambda i: (i, 0)),
            pl.BlockSpec((H, BT, D), lambda i: (0, i, 0)),
            pl.BlockSpec((H, NE * ES, D), lambda i: (0, 0, 0)),
            pl.BlockSpec((H, NE * ES, D), lambda i: (0, 0, 0)),
            pl.BlockSpec((E, 2 * E), lambda i: (0, 0)),
        ],
        out_specs=pl.BlockSpec((BT, E), lambda i: (i, 0)),
        out_shape=jax.ShapeDtypeStruct((T, E), F32),
    )(qf, gates, attn3, kf2, vf2, wout_bf)

    return out.reshape(B, T, E)
